# baseline (speedup 1.0000x reference)
import functools

import jax
import jax.numpy as jnp
from jax import lax
from jax.experimental import pallas as pl
from jax.experimental.pallas import tpu as pltpu
from jax.experimental.pallas import tpu_sc as plsc

_BATCH = 16384
_D = 64
_CLS = 100000
_WIN = 50048
_LO1 = 49920
_TAILBASE = _LO1 + _WIN
_NTAIL = _CLS - _TAILBASE
_NC = 2
_NS = 16
_NW = _NC * _NS
_DIMS_PW = _D // _NW
_NWAVE = _DIMS_PW * 2
_LANES = 16
_FCHUNK = 4096
_NFC = _BATCH // _FCHUNK
_UNROLL = 4

_mesh = plsc.VectorSubcoreMesh(core_axis_name="c", subcore_axis_name="s")


@functools.partial(
    pl.kernel,
    out_type=jax.ShapeDtypeStruct((_NW, _LANES), jnp.float32),
    mesh=_mesh,
    scratch_types=[
        pltpu.VMEM((_WIN,), jnp.float32),
        pltpu.VMEM((_WIN,), jnp.float32),
        pltpu.VMEM((_BATCH,), jnp.int32),
        pltpu.VMEM((_FCHUNK,), jnp.float32),
        pltpu.VMEM((_FCHUNK,), jnp.float32),
        pltpu.VMEM((_NTAIL,), jnp.float32),
        pltpu.VMEM((_NTAIL,), jnp.float32),
        pltpu.VMEM((_LANES,), jnp.float32),
        pltpu.SemaphoreType.DMA,
        pltpu.SemaphoreType.DMA,
        pltpu.SemaphoreType.DMA,
        pltpu.SemaphoreType.DMA,
    ],
    compiler_params=pltpu.CompilerParams(needs_layout_passes=False),
)
def _center_loss_partials(feat_hbm, lab_hbm, cent_hbm, tail_hbm, out_hbm,
                          crow0_v, crow1_v, lab_v, fb0_v, fb1_v,
                          tail0_v, tail1_v, acc_v,
                          csem0, csem1, fsem0, fsem1):
    wid = lax.axis_index("s") * _NC + lax.axis_index("c")

    crows = (crow0_v, crow1_v)
    csems = (csem0, csem1)
    fbufs = (fb0_v, fb1_v)
    fsems = (fsem0, fsem1)
    tails = (tail0_v, tail1_v)

    def wave_dim(w):
        return (w // 2) * _NW + wid

    def fire_crow(w):
        lo = (w % 2) * _LO1
        return pltpu.async_copy(
            cent_hbm.at[wave_dim(w)].at[pl.ds(lo, _WIN)],
            crows[w % 2], csems[w % 2])

    def fire_fchunk(w, fc):
        return pltpu.async_copy(
            feat_hbm.at[wave_dim(w), pl.ds(fc * _FCHUNK, _FCHUNK)],
            fbufs[fc % 2], fsems[fc % 2])

    cw = [fire_crow(0), fire_crow(1)]
    for t in range(_DIMS_PW):
        pltpu.sync_copy(tail_hbm.at[t * _NW + wid], tails[t])
    pltpu.sync_copy(lab_hbm, lab_v)

    accs = [jnp.zeros((_LANES,), jnp.float32) for _ in range(_UNROLL)]
    for w in range(_NWAVE):
        crow = crows[w % 2]
        tail = tails[w // 2]
        half = w % 2
        lo = jnp.int32(half * _LO1)
        cw[w % 2].wait()
        fw = fire_fchunk(w, 0)
        for fc in range(_NFC):
            fw.wait()
            fbuf = fbufs[fc % 2]
            if fc + 1 < _NFC:
                fw = fire_fchunk(w, fc + 1)
            elif w + 2 < _NWAVE:
                cw[w % 2] = fire_crow(w + 2)
            base = fc * _FCHUNK

            def blk(i, accs, crow=crow, fbuf=fbuf, base=base, lo=lo,
                    half=half, tail=tail):
                accs = list(accs)
                for u in range(_UNROLL):
                    o = (i * _UNROLL + u) * _LANES
                    raw = lab_v[pl.ds(base + o, _LANES)]
                    f = fbuf[pl.ds(o, _LANES)]
                    if half == 0:
                        m = raw < _WIN
                        c = plsc.load_gather(crow, [raw], mask=m)
                        df = jnp.where(m, f - c, 0.0)
                        accs[u] = accs[u] + df * df
                    else:
                        m = (raw >= _WIN) & (raw < _TAILBASE)
                        c = plsc.load_gather(crow, [raw - lo], mask=m)
                        mt = raw >= _TAILBASE
                        ct = plsc.load_gather(tail, [raw - _TAILBASE],
                                              mask=mt)
                        df = jnp.where(m, f - c, 0.0)
                        dft = jnp.where(mt, f - ct, 0.0)
                        accs[u] = accs[u] + df * df + dft * dft
                return tuple(accs)

            accs = lax.fori_loop(
                0, _FCHUNK // (_LANES * _UNROLL), blk, tuple(accs))
            accs = list(accs)

    acc_v[...] = (accs[0] + accs[1]) + (accs[2] + accs[3])
    pltpu.sync_copy(acc_v, out_hbm.at[wid])


def kernel(features, labels, centers):
    labels = labels.astype(jnp.int32)
    cent_t = centers.T
    tail = lax.slice(cent_t, (0, _TAILBASE), (_D, _CLS))
    partials = _center_loss_partials(features.T, labels, cent_t, tail)
    return jnp.sum(partials) / jnp.float32(_BATCH)


# speedup vs baseline: 1.1560x; 1.1560x over previous
import functools

import jax
import jax.numpy as jnp
from jax import lax
from jax.experimental import pallas as pl
from jax.experimental.pallas import tpu as pltpu
from jax.experimental.pallas import tpu_sc as plsc

_BATCH = 16384
_D = 64
_CLS = 100000
_NC = 2
_NS = 16
_NW = _NC * _NS
_WAVES = _D // _NW
_LANES = 16
_FCHUNK = 4096
_NFC = _BATCH // _FCHUNK
_UNROLL = 8

_mesh = plsc.VectorSubcoreMesh(core_axis_name="c", subcore_axis_name="s")


@functools.partial(
    pl.kernel,
    out_type=jax.ShapeDtypeStruct((_NW, _LANES), jnp.float32),
    mesh=_mesh,
    scratch_types=[
        pltpu.VMEM((_CLS,), jnp.float32),
        pltpu.VMEM((_BATCH,), jnp.int32),
        pltpu.VMEM((_FCHUNK,), jnp.float32),
        pltpu.VMEM((_FCHUNK,), jnp.float32),
        pltpu.VMEM((_LANES,), jnp.float32),
        pltpu.SemaphoreType.DMA,
        pltpu.SemaphoreType.DMA,
        pltpu.SemaphoreType.DMA,
    ],
    compiler_params=pltpu.CompilerParams(needs_layout_passes=False),
)
def _center_loss_partials(feat_hbm, lab_hbm, cent_hbm, out_hbm,
                          crow_v, lab_v, fb0_v, fb1_v, acc_v,
                          csem, fsem0, fsem1):
    wid = lax.axis_index("c") * _NS + lax.axis_index("s")

    fbufs = (fb0_v, fb1_v)
    fsems = (fsem0, fsem1)

    def fire_fchunk(d, fc):
        return pltpu.async_copy(
            feat_hbm.at[d, pl.ds(fc * _FCHUNK, _FCHUNK)],
            fbufs[fc % 2], fsems[fc % 2])

    d0 = wid * _WAVES
    cw = pltpu.async_copy(cent_hbm.at[d0], crow_v, csem)
    fw = fire_fchunk(d0, 0)
    pltpu.sync_copy(lab_hbm, lab_v)

    accs = [jnp.zeros((_LANES,), jnp.float32) for _ in range(_UNROLL)]
    for w in range(_WAVES):
        d = d0 + w
        cw.wait()
        for fc in range(_NFC):
            fw.wait()
            fbuf = fbufs[fc % 2]
            if fc + 1 < _NFC:
                fw = fire_fchunk(d, fc + 1)
            base = fc * _FCHUNK

            def blk(i, accs, fbuf=fbuf, base=base):
                accs = list(accs)
                for u in range(_UNROLL):
                    o = (i * _UNROLL + u) * _LANES
                    idx = lab_v[pl.ds(base + o, _LANES)]
                    c = plsc.load_gather(crow_v, [idx])
                    f = fbuf[pl.ds(o, _LANES)]
                    df = f - c
                    accs[u] = accs[u] + df * df
                return tuple(accs)

            accs = lax.fori_loop(
                0, _FCHUNK // (_LANES * _UNROLL), blk, tuple(accs))
            accs = list(accs)
        if w + 1 < _WAVES:
            cw = pltpu.async_copy(cent_hbm.at[d0 + w + 1], crow_v, csem)
            fw = fire_fchunk(d0 + w + 1, 0)

    r = accs[0]
    for u in range(1, _UNROLL):
        r = r + accs[u]
    acc_v[...] = r
    pltpu.sync_copy(acc_v, out_hbm.at[wid])


def kernel(features, labels, centers):
    labels = labels.astype(jnp.int32)
    partials = _center_loss_partials(features.T, labels, centers.T)
    return jnp.sum(partials) / jnp.float32(_BATCH)
